# Initial kernel scaffold; baseline (speedup 1.0000x reference)
import jax
import jax.numpy as jnp
from jax.experimental import pallas as pl


def kernel(h, edge_spk, edge_rep, edge_self, W_spk, al_spk, ar_spk, b_spk, W_rep, al_rep, ar_rep, b_rep, W_slf, al_slf, ar_slf, b_slf, Wp1, bp1, Wp2):
    raise NotImplementedError("write your pallas kernel here")


# speedup vs baseline: 24.7904x; 24.7904x over previous
import dataclasses
import functools

import jax
import jax.numpy as jnp
from jax import lax
from jax.experimental import pallas as pl
from jax.experimental.pallas import tpu as pltpu
from jax.experimental.pallas import tpu_sc as plsc

N = 10000
E = 160000
IN = 256
H = 4
D = 64
HD = H * D

NB = 10
BLK = N // NB
NS = 16
EPT = E // NS
CH = 80
NCHUNK = EPT // CH
NF = 10
FR = N // NF

_HIGH = jax.lax.Precision.HIGHEST

_GDN = lax.GatherDimensionNumbers(
    offset_dims=(), collapsed_slice_dims=(0,), start_index_map=(0,))


def _take16(vec, idx):
    return lax.gather(vec, idx[:, None], _GDN, (1,),
                      mode=lax.GatherScatterMode.PROMISE_IN_BOUNDS)


def _tc_a_body(h_ref, wcat_ref, alar_ref, feat_ref, eler_ref, bmax_ref):
    feat = jnp.dot(h_ref[...], wcat_ref[...],
                   preferred_element_type=jnp.float32, precision=_HIGH)
    feat_ref[...] = feat
    eler = jnp.dot(feat, alar_ref[...],
                   preferred_element_type=jnp.float32, precision=_HIGH)
    eler_ref[...] = eler
    bmax_ref[...] = jnp.max(eler, axis=0, keepdims=True)[None]


def _tc_a(h, wcat, alar):
    return pl.pallas_call(
        _tc_a_body,
        grid=(NB,),
        in_specs=[
            pl.BlockSpec((BLK, IN), lambda i: (i, 0)),
            pl.BlockSpec((IN, 3 * HD), lambda i: (0, 0)),
            pl.BlockSpec((3 * HD, 32), lambda i: (0, 0)),
        ],
        out_specs=[
            pl.BlockSpec((BLK, 3 * HD), lambda i: (i, 0)),
            pl.BlockSpec((BLK, 32), lambda i: (i, 0)),
            pl.BlockSpec((1, 1, 32), lambda i: (i, 0, 0)),
        ],
        out_shape=[
            jax.ShapeDtypeStruct((N, 3 * HD), jnp.float32),
            jax.ShapeDtypeStruct((N, 32), jnp.float32),
            jax.ShapeDtypeStruct((NB, 1, 32), jnp.float32),
        ],
    )(h, wcat, alar)


def _sc_body(feat_hbm, srcs_hbm, dsts_hbm, el_hbm, er_hbm, cpat_hbm,
             acc_hbm,
             acc_s, elv, erv, src_buf, dst_buf, rows, eebuf, cbuf):
    c = lax.axis_index("c")
    s = lax.axis_index("s")
    lane = lax.iota(jnp.int32, 16)
    zero16 = lane * 0
    zf16 = zero16.astype(jnp.float32)
    mask0 = jnp.where(lane == 0, 1.0, 0.0).astype(jnp.float32)

    for mp in range(3):
        for hh in range(2):
            t = (mp * 2 + c) * 2 + hh
            pltpu.sync_copy(el_hbm.at[pl.ds(t * N, N)], elv)
            pltpu.sync_copy(er_hbm.at[pl.ds(t * N, N)], erv)
            pltpu.sync_copy(cpat_hbm.at[pl.ds(t * 16, 16)], cbuf)

            @pl.loop(0, CH)
            def _(i):
                for v in range(8):
                    rows[i, pl.ds(v * 16, 16)] = zf16

            plsc.subcore_barrier()

            @pl.when(s < NF)
            def _():
                for z in range(12):
                    pltpu.sync_copy(rows,
                                    acc_s.at[pl.ds(s * FR + z * CH, CH)])
                pltpu.sync_copy(rows.at[pl.ds(0, FR - 12 * CH)],
                                acc_s.at[pl.ds(s * FR + 12 * CH, FR - 12 * CH)])
            plsc.subcore_barrier()

            cvec = cbuf[pl.ds(0, 16)]

            @pl.loop(0, NCHUNK)
            def _(k):
                start = mp * E + s * EPT + k * CH
                pltpu.sync_copy(srcs_hbm.at[pl.ds(start, CH)], src_buf)
                pltpu.sync_copy(dsts_hbm.at[pl.ds(start, CH)], dst_buf)
                pltpu.sync_copy(feat_hbm.at[mp, c, hh].at[src_buf], rows)

                @pl.loop(0, CH // 16)
                def _(j):
                    src16 = src_buf[pl.ds(j * 16, 16)]
                    dst16 = dst_buf[pl.ds(j * 16, 16)]
                    x = (plsc.load_gather(elv, [src16])
                         + plsc.load_gather(erv, [dst16]))
                    e = jnp.maximum(x, x * 0.2)
                    eebuf[pl.ds(j * 16, 16)] = jnp.exp(e - cvec)

                @pl.loop(0, CH // 16)
                def _(j):
                    eev = eebuf[pl.ds(j * 16, 16)]
                    for kk in range(16):
                        ed = j * 16 + kk
                        sk = _take16(eev, zero16 + kk)
                        for v in range(4):
                            rows[ed, pl.ds(v * 16, 16)] = (
                                rows[ed, pl.ds(v * 16, 16)] * sk)
                        rows[ed, pl.ds(64, 16)] = sk * mask0

                pltpu.sync_copy(rows, acc_s.at[dst_buf], add=True)

            plsc.subcore_barrier()

            @pl.when(s < NF)
            def _():
                pltpu.sync_copy(acc_s.at[pl.ds(s * FR, FR)],
                                acc_hbm.at[mp, c, hh].at[pl.ds(s * FR, FR)])
            plsc.subcore_barrier()


def _sc_run(feat_t, srcs, dsts, elt, ert, cpat):
    mesh = plsc.VectorSubcoreMesh(core_axis_name="c", subcore_axis_name="s")
    cp = pltpu.CompilerParams()
    if "needs_layout_passes" in pltpu.CompilerParams.__dataclass_fields__:
        cp = dataclasses.replace(cp, needs_layout_passes=False)
    f = pl.kernel(
        _sc_body,
        out_type=jax.ShapeDtypeStruct((3, 2, 2, N, 128), jnp.float32),
        mesh=mesh,
        scratch_types=[
            pltpu.VMEM_SHARED((N, 128), jnp.float32),
            pltpu.VMEM((N,), jnp.float32),
            pltpu.VMEM((N,), jnp.float32),
            pltpu.VMEM((CH,), jnp.int32),
            pltpu.VMEM((CH,), jnp.int32),
            pltpu.VMEM((CH, 128), jnp.float32),
            pltpu.VMEM((CH,), jnp.float32),
            pltpu.VMEM((16,), jnp.float32),
        ],
        compiler_params=cp,
    )
    return f(feat_t, srcs, dsts, elt, ert, cpat)


def _tc_b_body(acc_ref, bcat_ref, wp1_ref, bp1_ref, wp2_ref,
               zn_ref, wpart_ref):
    i = pl.program_id(0)

    @pl.when(i == 0)
    def _():
        wpart_ref[...] = jnp.zeros_like(wpart_ref)

    acc = acc_ref[...]
    bcat = bcat_ref[...]
    wp1 = wp1_ref[...]
    bp1 = bp1_ref[...]
    wp2 = wp2_ref[...]

    zmps = []
    wrows = []
    for mp in range(3):
        zs = []
        for c in range(2):
            for hh in range(2):
                a = acc[mp, c, hh, :, 0:D]
                d = acc[mp, c, hh, :, D:D + 1]
                dd = jnp.broadcast_to(d, (BLK, D))
                zs.append(jnp.where(dd > 0.0,
                                    a / jnp.where(dd > 0.0, dd, 1.0), 0.0))
        z = jnp.concatenate(zs, axis=1) + bcat[mp][None, :]
        zmps.append(z)
        t = jnp.tanh(jnp.dot(z, wp1, preferred_element_type=jnp.float32,
                             precision=_HIGH) + bp1)
        wrows.append(jnp.sum(t, axis=0, keepdims=True) * wp2)
    zn_ref[...] = jnp.stack(zmps, axis=0)
    wpart_ref[...] += jnp.concatenate(wrows, axis=0)


def _tc_b(acc, bcat, wp1, bp1, wp2):
    return pl.pallas_call(
        _tc_b_body,
        grid=(NB,),
        in_specs=[
            pl.BlockSpec((3, 2, 2, BLK, 128), lambda i: (0, 0, 0, i, 0)),
            pl.BlockSpec((3, HD), lambda i: (0, 0)),
            pl.BlockSpec((HD, 128), lambda i: (0, 0)),
            pl.BlockSpec((1, 128), lambda i: (0, 0)),
            pl.BlockSpec((1, 128), lambda i: (0, 0)),
        ],
        out_specs=[
            pl.BlockSpec((3, BLK, HD), lambda i: (0, i, 0)),
            pl.BlockSpec((3, 128), lambda i: (0, 0)),
        ],
        out_shape=[
            jax.ShapeDtypeStruct((3, N, HD), jnp.float32),
            jax.ShapeDtypeStruct((3, 128), jnp.float32),
        ],
    )(acc, bcat, wp1, bp1, wp2)


def _tc_c_body(zn_ref, wpart_ref, o_ref):
    wp = wpart_ref[...]
    w0 = jnp.sum(wp[0]) / N
    w1 = jnp.sum(wp[1]) / N
    w2 = jnp.sum(wp[2]) / N
    m = jnp.maximum(w0, jnp.maximum(w1, w2))
    b0 = jnp.exp(w0 - m)
    b1 = jnp.exp(w1 - m)
    b2 = jnp.exp(w2 - m)
    tot = b0 + b1 + b2
    zn = zn_ref[...]
    o_ref[...] = (b0 * zn[0] + b1 * zn[1] + b2 * zn[2]) / tot


def _tc_c(zn, wpart):
    return pl.pallas_call(
        _tc_c_body,
        grid=(NB,),
        in_specs=[
            pl.BlockSpec((3, BLK, HD), lambda i: (0, i, 0)),
            pl.BlockSpec((3, 128), lambda i: (0, 0)),
        ],
        out_specs=pl.BlockSpec((BLK, HD), lambda i: (i, 0)),
        out_shape=jax.ShapeDtypeStruct((N, HD), jnp.float32),
    )(zn, wpart)


def kernel(h, edge_spk, edge_rep, edge_self,
           W_spk, al_spk, ar_spk, b_spk,
           W_rep, al_rep, ar_rep, b_rep,
           W_slf, al_slf, ar_slf, b_slf,
           Wp1, bp1, Wp2):
    f32 = jnp.float32

    wcat = jnp.concatenate([W_spk, W_rep, W_slf], axis=1)
    al_all = jnp.concatenate([al_spk, al_rep, al_slf], axis=0)
    ar_all = jnp.concatenate([ar_spk, ar_rep, ar_slf], axis=0)
    eye = jnp.eye(12, dtype=f32)
    al_mat = (eye[:, None, :] * al_all[:, :, None]).reshape(3 * HD, 12)
    ar_mat = (eye[:, None, :] * ar_all[:, :, None]).reshape(3 * HD, 12)
    pad4 = jnp.zeros((3 * HD, 4), f32)
    alar = jnp.concatenate([al_mat, pad4, ar_mat, pad4], axis=1)

    feat, eler, bmax = _tc_a(h, wcat, alar)

    feat_r = feat.reshape(N, 3, 2, 2, D)
    feat_r = jnp.concatenate(
        [feat_r, jnp.zeros((N, 3, 2, 2, 128 - D), f32)], axis=-1)
    feat_t = feat_r.transpose(1, 2, 3, 0, 4)
    el_t = eler[:, 0:12].reshape(N, 3, 2, 2).transpose(1, 2, 3, 0)
    el_t = el_t.reshape(3 * 2 * 2 * N)
    er_t = eler[:, 16:28].reshape(N, 3, 2, 2).transpose(1, 2, 3, 0)
    er_t = er_t.reshape(3 * 2 * 2 * N)
    bm = jnp.max(bmax.reshape(NB, 32), axis=0)
    msum = bm[0:12] + bm[16:28]
    c12 = jnp.maximum(msum, 0.2 * msum).reshape(3, 2, 2, 1)
    cpat = jnp.broadcast_to(c12, (3, 2, 2, 16)).reshape(3 * 2 * 2 * 16)
    srcs = jnp.concatenate([edge_spk[0], edge_rep[0], edge_self[0]])
    dsts = jnp.concatenate([edge_spk[1], edge_rep[1], edge_self[1]])

    acc = _sc_run(feat_t, srcs, dsts, el_t, er_t, cpat)

    bcat = jnp.stack([b_spk, b_rep, b_slf])
    zn, wpart = _tc_b(acc, bcat, Wp1, bp1.reshape(1, 128),
                      Wp2.reshape(1, 128))
    return _tc_c(zn, wpart)


# speedup vs baseline: 44.2555x; 1.7852x over previous
import dataclasses
import functools

import jax
import jax.numpy as jnp
from jax import lax
from jax.experimental import pallas as pl
from jax.experimental.pallas import tpu as pltpu
from jax.experimental.pallas import tpu_sc as plsc

N = 10000
E = 160000
IN = 256
H = 4
D = 64
HD = H * D

NB = 10
BLK = N // NB
NS = 16
EPT = E // NS
CH = 80
NCHUNK = EPT // CH
NF = 10
FR = N // NF

_HIGH = jax.lax.Precision.HIGHEST

_GDN = lax.GatherDimensionNumbers(
    offset_dims=(), collapsed_slice_dims=(0,), start_index_map=(0,))


def _take16(vec, idx):
    return lax.gather(vec, idx[:, None], _GDN, (1,),
                      mode=lax.GatherScatterMode.PROMISE_IN_BOUNDS)


def _tc_a_body(h_ref, wcat_ref, alar_ref, feat_ref, eler_ref, bmax_ref):
    feat = jnp.dot(h_ref[...], wcat_ref[...],
                   preferred_element_type=jnp.float32, precision=_HIGH)
    feat_ref[...] = feat
    eler = jnp.dot(feat, alar_ref[...],
                   preferred_element_type=jnp.float32, precision=_HIGH)
    eler_ref[...] = eler
    bmax_ref[...] = jnp.max(eler, axis=0, keepdims=True)[None]


def _tc_a(h, wcat, alar):
    return pl.pallas_call(
        _tc_a_body,
        grid=(NB,),
        in_specs=[
            pl.BlockSpec((BLK, IN), lambda i: (i, 0)),
            pl.BlockSpec((IN, 3 * HD), lambda i: (0, 0)),
            pl.BlockSpec((3 * HD, 32), lambda i: (0, 0)),
        ],
        out_specs=[
            pl.BlockSpec((BLK, 3 * HD), lambda i: (i, 0)),
            pl.BlockSpec((BLK, 32), lambda i: (i, 0)),
            pl.BlockSpec((1, 1, 32), lambda i: (i, 0, 0)),
        ],
        out_shape=[
            jax.ShapeDtypeStruct((N, 3 * HD), jnp.float32),
            jax.ShapeDtypeStruct((N, 32), jnp.float32),
            jax.ShapeDtypeStruct((NB, 1, 32), jnp.float32),
        ],
    )(h, wcat, alar)


def _sc_body(feat_hbm, srcs_hbm, dsts_hbm, el_hbm, er_hbm, cpat_hbm,
             acc_hbm,
             acc_s, elv, erv, src0, dst0, src1, dst1, rows0, rows1, eebuf,
             cbuf, sg0, sg1, si0, si1):
    c = lax.axis_index("c")
    s = lax.axis_index("s")
    lane = lax.iota(jnp.int32, 16)
    zero16 = lane * 0
    zf16 = zero16.astype(jnp.float32)
    mask0 = jnp.where(lane == 0, 1.0, 0.0).astype(jnp.float32)

    for mp in range(3):
        for hh in range(2):
            t = (mp * 2 + c) * 2 + hh
            base = mp * E + s * EPT
            pltpu.sync_copy(el_hbm.at[pl.ds(t * N, N)], elv)
            pltpu.sync_copy(er_hbm.at[pl.ds(t * N, N)], erv)
            pltpu.sync_copy(cpat_hbm.at[pl.ds(t * 16, 16)], cbuf)

            @pl.loop(0, CH)
            def _(i):
                for v in range(8):
                    rows0[i, pl.ds(v * 16, 16)] = zf16

            plsc.subcore_barrier()

            @pl.when(s < NF)
            def _():
                for z in range(12):
                    pltpu.sync_copy(rows0,
                                    acc_s.at[pl.ds(s * FR + z * CH, CH)])
                pltpu.sync_copy(rows0.at[pl.ds(0, FR - 12 * CH)],
                                acc_s.at[pl.ds(s * FR + 12 * CH, FR - 12 * CH)])
            plsc.subcore_barrier()

            cvec = cbuf[pl.ds(0, 16)]

            def idx_start(k, sb, db, sem):
                st = base + k * CH
                pltpu.async_copy(srcs_hbm.at[pl.ds(st, CH)], sb, sem)
                pltpu.async_copy(dsts_hbm.at[pl.ds(st, CH)], db, sem)

            def idx_wait(sb, db, sem):
                pltpu.make_async_copy(srcs_hbm.at[pl.ds(0, CH)], sb, sem).wait()
                pltpu.make_async_copy(dsts_hbm.at[pl.ds(0, CH)], db, sem).wait()

            def gather_start(sb, rows, sem):
                pltpu.async_copy(feat_hbm.at[mp, c, hh].at[sb], rows, sem)

            def gather_wait(sb, rows, sem):
                pltpu.make_async_copy(
                    feat_hbm.at[mp, c, hh].at[sb], rows, sem).wait()

            def compute_ee(sb, db):
                @pl.loop(0, CH // 16)
                def _(j):
                    src16 = sb[pl.ds(j * 16, 16)]
                    dst16 = db[pl.ds(j * 16, 16)]
                    x = (plsc.load_gather(elv, [src16])
                         + plsc.load_gather(erv, [dst16]))
                    e = jnp.maximum(x, x * 0.2)
                    eebuf[pl.ds(j * 16, 16)] = jnp.exp(e - cvec)

            def scale_rows(rows):
                @pl.loop(0, CH // 16)
                def _(j):
                    eev = eebuf[pl.ds(j * 16, 16)]
                    for kk in range(16):
                        ed = j * 16 + kk
                        sk = _take16(eev, zero16 + kk)
                        for v in range(4):
                            rows[ed, pl.ds(v * 16, 16)] = (
                                rows[ed, pl.ds(v * 16, 16)] * sk)
                        rows[ed, pl.ds(64, 16)] = sk * mask0

            idx_start(0, src0, dst0, si0)
            idx_wait(src0, dst0, si0)
            idx_start(1, src1, dst1, si1)
            gather_start(src0, rows0, sg0)

            @pl.loop(0, NCHUNK // 2)
            def _(i):
                idx_wait(src1, dst1, si1)
                gather_start(src1, rows1, sg1)
                compute_ee(src0, dst0)
                gather_wait(src0, rows0, sg0)
                scale_rows(rows0)
                pltpu.sync_copy(rows0, acc_s.at[dst0], add=True)
                idx_start(2 * i + 2, src0, dst0, si0)
                compute_ee(src1, dst1)
                gather_wait(src1, rows1, sg1)
                scale_rows(rows1)
                pltpu.sync_copy(rows1, acc_s.at[dst1], add=True)
                idx_wait(src0, dst0, si0)
                gather_start(src0, rows0, sg0)

                @pl.when(2 * i + 3 < NCHUNK)
                def _():
                    idx_start(2 * i + 3, src1, dst1, si1)

            compute_ee(src0, dst0)
            gather_wait(src0, rows0, sg0)
            scale_rows(rows0)
            pltpu.sync_copy(rows0, acc_s.at[dst0], add=True)

            plsc.subcore_barrier()

            @pl.when(s < NF)
            def _():
                pltpu.sync_copy(acc_s.at[pl.ds(s * FR, FR)],
                                acc_hbm.at[mp, c, hh].at[pl.ds(s * FR, FR)])
            plsc.subcore_barrier()


def _sc_run(feat_t, srcs, dsts, elt, ert, cpat):
    mesh = plsc.VectorSubcoreMesh(core_axis_name="c", subcore_axis_name="s")
    cp = pltpu.CompilerParams()
    if "needs_layout_passes" in pltpu.CompilerParams.__dataclass_fields__:
        cp = dataclasses.replace(cp, needs_layout_passes=False)
    f = pl.kernel(
        _sc_body,
        out_type=jax.ShapeDtypeStruct((3, 2, 2, N, 128), jnp.float32),
        mesh=mesh,
        scratch_types=[
            pltpu.VMEM_SHARED((N, 128), jnp.float32),
            pltpu.VMEM((N,), jnp.float32),
            pltpu.VMEM((N,), jnp.float32),
            pltpu.VMEM((CH,), jnp.int32),
            pltpu.VMEM((CH,), jnp.int32),
            pltpu.VMEM((CH,), jnp.int32),
            pltpu.VMEM((CH,), jnp.int32),
            pltpu.VMEM((CH, 128), jnp.float32),
            pltpu.VMEM((CH, 128), jnp.float32),
            pltpu.VMEM((CH,), jnp.float32),
            pltpu.VMEM((16,), jnp.float32),
            pltpu.SemaphoreType.DMA,
            pltpu.SemaphoreType.DMA,
            pltpu.SemaphoreType.DMA,
            pltpu.SemaphoreType.DMA,
        ],
        compiler_params=cp,
    )
    return f(feat_t, srcs, dsts, elt, ert, cpat)


def _tc_b_body(acc_ref, bcat_ref, wp1_ref, bp1_ref, wp2_ref,
               zn_ref, wpart_ref):
    i = pl.program_id(0)

    @pl.when(i == 0)
    def _():
        wpart_ref[...] = jnp.zeros_like(wpart_ref)

    acc = acc_ref[...]
    bcat = bcat_ref[...]
    wp1 = wp1_ref[...]
    bp1 = bp1_ref[...]
    wp2 = wp2_ref[...]

    zmps = []
    wrows = []
    for mp in range(3):
        zs = []
        for c in range(2):
            for hh in range(2):
                a = acc[mp, c, hh, :, 0:D]
                d = acc[mp, c, hh, :, D:D + 1]
                dd = jnp.broadcast_to(d, (BLK, D))
                zs.append(jnp.where(dd > 0.0,
                                    a / jnp.where(dd > 0.0, dd, 1.0), 0.0))
        z = jnp.concatenate(zs, axis=1) + bcat[mp][None, :]
        zmps.append(z)
        t = jnp.tanh(jnp.dot(z, wp1, preferred_element_type=jnp.float32,
                             precision=_HIGH) + bp1)
        wrows.append(jnp.sum(t, axis=0, keepdims=True) * wp2)
    zn_ref[...] = jnp.stack(zmps, axis=0)
    wpart_ref[...] += jnp.concatenate(wrows, axis=0)


def _tc_b(acc, bcat, wp1, bp1, wp2):
    return pl.pallas_call(
        _tc_b_body,
        grid=(NB,),
        in_specs=[
            pl.BlockSpec((3, 2, 2, BLK, 128), lambda i: (0, 0, 0, i, 0)),
            pl.BlockSpec((3, HD), lambda i: (0, 0)),
            pl.BlockSpec((HD, 128), lambda i: (0, 0)),
            pl.BlockSpec((1, 128), lambda i: (0, 0)),
            pl.BlockSpec((1, 128), lambda i: (0, 0)),
        ],
        out_specs=[
            pl.BlockSpec((3, BLK, HD), lambda i: (0, i, 0)),
            pl.BlockSpec((3, 128), lambda i: (0, 0)),
        ],
        out_shape=[
            jax.ShapeDtypeStruct((3, N, HD), jnp.float32),
            jax.ShapeDtypeStruct((3, 128), jnp.float32),
        ],
    )(acc, bcat, wp1, bp1, wp2)


def _tc_c_body(zn_ref, wpart_ref, o_ref):
    wp = wpart_ref[...]
    w0 = jnp.sum(wp[0]) / N
    w1 = jnp.sum(wp[1]) / N
    w2 = jnp.sum(wp[2]) / N
    m = jnp.maximum(w0, jnp.maximum(w1, w2))
    b0 = jnp.exp(w0 - m)
    b1 = jnp.exp(w1 - m)
    b2 = jnp.exp(w2 - m)
    tot = b0 + b1 + b2
    zn = zn_ref[...]
    o_ref[...] = (b0 * zn[0] + b1 * zn[1] + b2 * zn[2]) / tot


def _tc_c(zn, wpart):
    return pl.pallas_call(
        _tc_c_body,
        grid=(NB,),
        in_specs=[
            pl.BlockSpec((3, BLK, HD), lambda i: (0, i, 0)),
            pl.BlockSpec((3, 128), lambda i: (0, 0)),
        ],
        out_specs=pl.BlockSpec((BLK, HD), lambda i: (i, 0)),
        out_shape=jax.ShapeDtypeStruct((N, HD), jnp.float32),
    )(zn, wpart)


def kernel(h, edge_spk, edge_rep, edge_self,
           W_spk, al_spk, ar_spk, b_spk,
           W_rep, al_rep, ar_rep, b_rep,
           W_slf, al_slf, ar_slf, b_slf,
           Wp1, bp1, Wp2):
    f32 = jnp.float32

    wcat = jnp.concatenate([W_spk, W_rep, W_slf], axis=1)
    al_all = jnp.concatenate([al_spk, al_rep, al_slf], axis=0)
    ar_all = jnp.concatenate([ar_spk, ar_rep, ar_slf], axis=0)
    eye = jnp.eye(12, dtype=f32)
    al_mat = (eye[:, None, :] * al_all[:, :, None]).reshape(3 * HD, 12)
    ar_mat = (eye[:, None, :] * ar_all[:, :, None]).reshape(3 * HD, 12)
    pad4 = jnp.zeros((3 * HD, 4), f32)
    alar = jnp.concatenate([al_mat, pad4, ar_mat, pad4], axis=1)

    feat, eler, bmax = _tc_a(h, wcat, alar)

    feat_r = feat.reshape(N, 3, 2, 2, D)
    feat_r = jnp.concatenate(
        [feat_r, jnp.zeros((N, 3, 2, 2, 128 - D), f32)], axis=-1)
    feat_t = feat_r.transpose(1, 2, 3, 0, 4)
    el_t = eler[:, 0:12].reshape(N, 3, 2, 2).transpose(1, 2, 3, 0)
    el_t = el_t.reshape(3 * 2 * 2 * N)
    er_t = eler[:, 16:28].reshape(N, 3, 2, 2).transpose(1, 2, 3, 0)
    er_t = er_t.reshape(3 * 2 * 2 * N)
    bm = jnp.max(bmax.reshape(NB, 32), axis=0)
    msum = bm[0:12] + bm[16:28]
    c12 = jnp.maximum(msum, 0.2 * msum).reshape(3, 2, 2, 1)
    cpat = jnp.broadcast_to(c12, (3, 2, 2, 16)).reshape(3 * 2 * 2 * 16)
    srcs = jnp.concatenate([edge_spk[0], edge_rep[0], edge_self[0]])
    dsts = jnp.concatenate([edge_spk[1], edge_rep[1], edge_self[1]])

    acc = _sc_run(feat_t, srcs, dsts, el_t, er_t, cpat)

    bcat = jnp.stack([b_spk, b_rep, b_slf])
    zn, wpart = _tc_b(acc, bcat, Wp1, bp1.reshape(1, 128),
                      Wp2.reshape(1, 128))
    return _tc_c(zn, wpart)


# speedup vs baseline: 53.4857x; 1.2086x over previous
import dataclasses
import functools

import jax
import jax.numpy as jnp
from jax import lax
from jax.experimental import pallas as pl
from jax.experimental.pallas import tpu as pltpu
from jax.experimental.pallas import tpu_sc as plsc

N = 10000
E = 160000
IN = 256
H = 4
D = 64
HD = H * D

NB = 10
BLK = N // NB
NS = 16
EPT = E // NS
CH = 80
NCHUNK = EPT // CH
NF = 10
FR = N // NF

_HIGH = jax.lax.Precision.HIGHEST

_GDN = lax.GatherDimensionNumbers(
    offset_dims=(), collapsed_slice_dims=(0,), start_index_map=(0,))


def _take16(vec, idx):
    return lax.gather(vec, idx[:, None], _GDN, (1,),
                      mode=lax.GatherScatterMode.PROMISE_IN_BOUNDS)


def _tc_a_body(h_ref, wcat_ref, alar_ref, feat_ref, eler_ref, bmax_ref):
    feat = jnp.dot(h_ref[...], wcat_ref[...],
                   preferred_element_type=jnp.float32, precision=_HIGH)
    feat_ref[...] = feat
    eler = jnp.dot(feat, alar_ref[...],
                   preferred_element_type=jnp.float32, precision=_HIGH)
    eler_ref[...] = eler
    bmax_ref[...] = jnp.max(eler, axis=0, keepdims=True)[None]


def _tc_a(h, wcat, alar):
    return pl.pallas_call(
        _tc_a_body,
        grid=(NB,),
        in_specs=[
            pl.BlockSpec((BLK, IN), lambda i: (i, 0)),
            pl.BlockSpec((IN, 3 * HD), lambda i: (0, 0)),
            pl.BlockSpec((3 * HD, 32), lambda i: (0, 0)),
        ],
        out_specs=[
            pl.BlockSpec((BLK, 3 * HD), lambda i: (i, 0)),
            pl.BlockSpec((BLK, 32), lambda i: (i, 0)),
            pl.BlockSpec((1, 1, 32), lambda i: (i, 0, 0)),
        ],
        out_shape=[
            jax.ShapeDtypeStruct((N, 3 * HD), jnp.float32),
            jax.ShapeDtypeStruct((N, 32), jnp.float32),
            jax.ShapeDtypeStruct((NB, 1, 32), jnp.float32),
        ],
    )(h, wcat, alar)


def _sc_body(feat_hbm, srcs_hbm, dsts_hbm, el_hbm, er_hbm, cpat_hbm,
             acc_hbm,
             acc_s, elv, erv, src0, dst0, src1, dst1, dstS0, dstS1,
             rows0, rows1, eebuf, cbuf, sg0, sg1, si0, si1, semS0, semS1):
    c = lax.axis_index("c")
    s = lax.axis_index("s")
    lane = lax.iota(jnp.int32, 16)
    zero16 = lane * 0
    zf16 = zero16.astype(jnp.float32)
    mask0 = jnp.where(lane == 0, 1.0, 0.0).astype(jnp.float32)

    for mp in range(3):
        for hh in range(2):
            t = (mp * 2 + c) * 2 + hh
            base = mp * E + s * EPT
            pltpu.sync_copy(el_hbm.at[pl.ds(t * N, N)], elv)
            pltpu.sync_copy(er_hbm.at[pl.ds(t * N, N)], erv)
            pltpu.sync_copy(cpat_hbm.at[pl.ds(t * 16, 16)], cbuf)

            @pl.loop(0, CH)
            def _(i):
                for v in range(8):
                    rows0[i, pl.ds(v * 16, 16)] = zf16

            plsc.subcore_barrier()

            @pl.when(s < NF)
            def _():
                for z in range(12):
                    pltpu.sync_copy(rows0,
                                    acc_s.at[pl.ds(s * FR + z * CH, CH)])
                pltpu.sync_copy(rows0.at[pl.ds(0, FR - 12 * CH)],
                                acc_s.at[pl.ds(s * FR + 12 * CH, FR - 12 * CH)])
            plsc.subcore_barrier()

            cvec = cbuf[pl.ds(0, 16)]

            def idx_start(k, sb, db, sem):
                st = base + k * CH
                pltpu.async_copy(srcs_hbm.at[pl.ds(st, CH)], sb, sem)
                pltpu.async_copy(dsts_hbm.at[pl.ds(st, CH)], db, sem)

            def idx_wait(sb, db, sem):
                pltpu.make_async_copy(srcs_hbm.at[pl.ds(0, CH)], sb, sem).wait()
                pltpu.make_async_copy(dsts_hbm.at[pl.ds(0, CH)], db, sem).wait()

            def gather_start(sb, rows, sem):
                pltpu.async_copy(feat_hbm.at[mp, c, hh].at[sb], rows, sem)

            def gather_wait(sb, rows, sem):
                pltpu.make_async_copy(
                    feat_hbm.at[mp, c, hh].at[sb], rows, sem).wait()

            def compute_ee(sb, db):
                @pl.loop(0, CH // 16)
                def _(j):
                    src16 = sb[pl.ds(j * 16, 16)]
                    dst16 = db[pl.ds(j * 16, 16)]
                    x = (plsc.load_gather(elv, [src16])
                         + plsc.load_gather(erv, [dst16]))
                    e = jnp.maximum(x, x * 0.2)
                    eebuf[pl.ds(j * 16, 16)] = jnp.exp(e - cvec)

            def scale_rows(rows):
                @pl.loop(0, CH // 16)
                def _(j):
                    eev = eebuf[pl.ds(j * 16, 16)]
                    for kk in range(16):
                        ed = j * 16 + kk
                        sk = _take16(eev, zero16 + kk)
                        for v in range(4):
                            rows[ed, pl.ds(v * 16, 16)] = (
                                rows[ed, pl.ds(v * 16, 16)] * sk)
                        rows[ed, pl.ds(64, 16)] = sk * mask0

            def copy_idx(db, dbs):
                @pl.loop(0, CH // 16)
                def _(j):
                    dbs[pl.ds(j * 16, 16)] = db[pl.ds(j * 16, 16)]

            def scatter_start(rows, dbs, sem):
                pltpu.async_copy(rows, acc_s.at[dbs], sem, add=True)

            def scatter_wait(rows, dbs, sem):
                pltpu.make_async_copy(rows, acc_s.at[dbs], sem).wait()

            idx_start(0, src0, dst0, si0)
            idx_wait(src0, dst0, si0)
            idx_start(1, src1, dst1, si1)
            idx_wait(src1, dst1, si1)
            gather_start(src0, rows0, sg0)

            @pl.loop(0, NCHUNK // 2)
            def _(i):
                compute_ee(src0, dst0)

                @pl.when(i > 0)
                def _():
                    scatter_wait(rows1, dstS1, semS1)
                    idx_wait(src1, dst1, si1)
                gather_start(src1, rows1, sg1)
                gather_wait(src0, rows0, sg0)
                scale_rows(rows0)
                copy_idx(dst0, dstS0)
                scatter_start(rows0, dstS0, semS0)
                idx_start(2 * i + 2, src0, dst0, si0)
                compute_ee(src1, dst1)
                scatter_wait(rows0, dstS0, semS0)
                idx_wait(src0, dst0, si0)
                gather_start(src0, rows0, sg0)
                gather_wait(src1, rows1, sg1)
                scale_rows(rows1)
                copy_idx(dst1, dstS1)
                scatter_start(rows1, dstS1, semS1)

                @pl.when(2 * i + 3 < NCHUNK)
                def _():
                    idx_start(2 * i + 3, src1, dst1, si1)

            scatter_wait(rows1, dstS1, semS1)
            compute_ee(src0, dst0)
            gather_wait(src0, rows0, sg0)
            scale_rows(rows0)
            copy_idx(dst0, dstS0)
            scatter_start(rows0, dstS0, semS0)
            scatter_wait(rows0, dstS0, semS0)

            plsc.subcore_barrier()

            @pl.when(s < NF)
            def _():
                pltpu.sync_copy(acc_s.at[pl.ds(s * FR, FR)],
                                acc_hbm.at[mp, c, hh].at[pl.ds(s * FR, FR)])
            plsc.subcore_barrier()


def _sc_run(feat_t, srcs, dsts, elt, ert, cpat):
    mesh = plsc.VectorSubcoreMesh(core_axis_name="c", subcore_axis_name="s")
    cp = pltpu.CompilerParams()
    if "needs_layout_passes" in pltpu.CompilerParams.__dataclass_fields__:
        cp = dataclasses.replace(cp, needs_layout_passes=False)
    f = pl.kernel(
        _sc_body,
        out_type=jax.ShapeDtypeStruct((3, 2, 2, N, 128), jnp.float32),
        mesh=mesh,
        scratch_types=[
            pltpu.VMEM_SHARED((N, 128), jnp.float32),
            pltpu.VMEM((N,), jnp.float32),
            pltpu.VMEM((N,), jnp.float32),
            pltpu.VMEM((CH,), jnp.int32),
            pltpu.VMEM((CH,), jnp.int32),
            pltpu.VMEM((CH,), jnp.int32),
            pltpu.VMEM((CH,), jnp.int32),
            pltpu.VMEM((CH,), jnp.int32),
            pltpu.VMEM((CH,), jnp.int32),
            pltpu.VMEM((CH, 128), jnp.float32),
            pltpu.VMEM((CH, 128), jnp.float32),
            pltpu.VMEM((CH,), jnp.float32),
            pltpu.VMEM((16,), jnp.float32),
            pltpu.SemaphoreType.DMA,
            pltpu.SemaphoreType.DMA,
            pltpu.SemaphoreType.DMA,
            pltpu.SemaphoreType.DMA,
            pltpu.SemaphoreType.DMA,
            pltpu.SemaphoreType.DMA,
        ],
        compiler_params=cp,
    )
    return f(feat_t, srcs, dsts, elt, ert, cpat)


def _tc_b_body(acc_ref, bcat_ref, wp1_ref, bp1_ref, wp2_ref,
               zn_ref, wpart_ref):
    i = pl.program_id(0)

    @pl.when(i == 0)
    def _():
        wpart_ref[...] = jnp.zeros_like(wpart_ref)

    acc = acc_ref[...]
    bcat = bcat_ref[...]
    wp1 = wp1_ref[...]
    bp1 = bp1_ref[...]
    wp2 = wp2_ref[...]

    zmps = []
    wrows = []
    for mp in range(3):
        zs = []
        for c in range(2):
            for hh in range(2):
                a = acc[mp, c, hh, :, 0:D]
                d = acc[mp, c, hh, :, D:D + 1]
                dd = jnp.broadcast_to(d, (BLK, D))
                zs.append(jnp.where(dd > 0.0,
                                    a / jnp.where(dd > 0.0, dd, 1.0), 0.0))
        z = jnp.concatenate(zs, axis=1) + bcat[mp][None, :]
        zmps.append(z)
        t = jnp.tanh(jnp.dot(z, wp1, preferred_element_type=jnp.float32,
                             precision=_HIGH) + bp1)
        wrows.append(jnp.sum(t, axis=0, keepdims=True) * wp2)
    zn_ref[...] = jnp.stack(zmps, axis=0)
    wpart_ref[...] += jnp.concatenate(wrows, axis=0)


def _tc_b(acc, bcat, wp1, bp1, wp2):
    return pl.pallas_call(
        _tc_b_body,
        grid=(NB,),
        in_specs=[
            pl.BlockSpec((3, 2, 2, BLK, 128), lambda i: (0, 0, 0, i, 0)),
            pl.BlockSpec((3, HD), lambda i: (0, 0)),
            pl.BlockSpec((HD, 128), lambda i: (0, 0)),
            pl.BlockSpec((1, 128), lambda i: (0, 0)),
            pl.BlockSpec((1, 128), lambda i: (0, 0)),
        ],
        out_specs=[
            pl.BlockSpec((3, BLK, HD), lambda i: (0, i, 0)),
            pl.BlockSpec((3, 128), lambda i: (0, 0)),
        ],
        out_shape=[
            jax.ShapeDtypeStruct((3, N, HD), jnp.float32),
            jax.ShapeDtypeStruct((3, 128), jnp.float32),
        ],
    )(acc, bcat, wp1, bp1, wp2)


def _tc_c_body(zn_ref, wpart_ref, o_ref):
    wp = wpart_ref[...]
    w0 = jnp.sum(wp[0]) / N
    w1 = jnp.sum(wp[1]) / N
    w2 = jnp.sum(wp[2]) / N
    m = jnp.maximum(w0, jnp.maximum(w1, w2))
    b0 = jnp.exp(w0 - m)
    b1 = jnp.exp(w1 - m)
    b2 = jnp.exp(w2 - m)
    tot = b0 + b1 + b2
    zn = zn_ref[...]
    o_ref[...] = (b0 * zn[0] + b1 * zn[1] + b2 * zn[2]) / tot


def _tc_c(zn, wpart):
    return pl.pallas_call(
        _tc_c_body,
        grid=(NB,),
        in_specs=[
            pl.BlockSpec((3, BLK, HD), lambda i: (0, i, 0)),
            pl.BlockSpec((3, 128), lambda i: (0, 0)),
        ],
        out_specs=pl.BlockSpec((BLK, HD), lambda i: (i, 0)),
        out_shape=jax.ShapeDtypeStruct((N, HD), jnp.float32),
    )(zn, wpart)


def kernel(h, edge_spk, edge_rep, edge_self,
           W_spk, al_spk, ar_spk, b_spk,
           W_rep, al_rep, ar_rep, b_rep,
           W_slf, al_slf, ar_slf, b_slf,
           Wp1, bp1, Wp2):
    f32 = jnp.float32

    wcat = jnp.concatenate([W_spk, W_rep, W_slf], axis=1)
    al_all = jnp.concatenate([al_spk, al_rep, al_slf], axis=0)
    ar_all = jnp.concatenate([ar_spk, ar_rep, ar_slf], axis=0)
    eye = jnp.eye(12, dtype=f32)
    al_mat = (eye[:, None, :] * al_all[:, :, None]).reshape(3 * HD, 12)
    ar_mat = (eye[:, None, :] * ar_all[:, :, None]).reshape(3 * HD, 12)
    pad4 = jnp.zeros((3 * HD, 4), f32)
    alar = jnp.concatenate([al_mat, pad4, ar_mat, pad4], axis=1)

    feat, eler, bmax = _tc_a(h, wcat, alar)

    feat_r = feat.reshape(N, 3, 2, 2, D)
    feat_r = jnp.concatenate(
        [feat_r, jnp.zeros((N, 3, 2, 2, 128 - D), f32)], axis=-1)
    feat_t = feat_r.transpose(1, 2, 3, 0, 4)
    el_t = eler[:, 0:12].reshape(N, 3, 2, 2).transpose(1, 2, 3, 0)
    el_t = el_t.reshape(3 * 2 * 2 * N)
    er_t = eler[:, 16:28].reshape(N, 3, 2, 2).transpose(1, 2, 3, 0)
    er_t = er_t.reshape(3 * 2 * 2 * N)
    bm = jnp.max(bmax.reshape(NB, 32), axis=0)
    msum = bm[0:12] + bm[16:28]
    c12 = jnp.maximum(msum, 0.2 * msum).reshape(3, 2, 2, 1)
    cpat = jnp.broadcast_to(c12, (3, 2, 2, 16)).reshape(3 * 2 * 2 * 16)
    srcs = jnp.concatenate([edge_spk[0], edge_rep[0], edge_self[0]])
    dsts = jnp.concatenate([edge_spk[1], edge_rep[1], edge_self[1]])

    acc = _sc_run(feat_t, srcs, dsts, el_t, er_t, cpat)

    bcat = jnp.stack([b_spk, b_rep, b_slf])
    zn, wpart = _tc_b(acc, bcat, Wp1, bp1.reshape(1, 128),
                      Wp2.reshape(1, 128))
    return _tc_c(zn, wpart)
